# baseline (speedup 1.0000x reference)
import functools

import jax
import jax.numpy as jnp
from jax import lax
from jax.experimental import pallas as pl
from jax.experimental.pallas import tpu as pltpu
from jax.experimental.pallas import tpu_sc as plsc

S1, S2 = 8192, 4096
R = 16
K = 8

_info = plsc.get_sparse_core_info()
_NC, _NS = _info.num_cores, _info.num_subcores
_NW = _NC * _NS


def _interleave(kt_v, i_v, n_out, w):
    def step(t, carry):
        base = t * R
        jv = base + lax.iota(jnp.int32, R)
        src = (jv & (K - 1)) * w + lax.shift_right_logical(jv, 3)
        i_v[pl.ds(base, R)] = plsc.load_gather(kt_v, [src])
        return carry

    lax.fori_loop(0, n_out // R, step, 0)


def _make_sc_gather(s_rows):
    n_out = s_rows * K
    c = n_out // _NW
    w = s_rows // _NW
    h = c // 2

    def body(i_hbm, m_hbm, o_hbm, m_sh, k_v, i_v, r_v, sem, sem2, sem3):
        sid = lax.axis_index("s")
        wid = sid * _NC + lax.axis_index("c")
        b = wid * c
        wb = wid * w

        @pl.when(sid == 0)
        def _():
            pltpu.sync_copy(m_hbm, m_sh)

        cps = [pltpu.async_copy(i_hbm.at[pl.ds(k * s_rows + wb, w)],
                                k_v.at[pl.ds(k * w, w)], sem3)
               for k in range(K)]
        for cp in cps:
            cp.wait()
        _interleave(k_v, i_v, c, w)
        plsc.subcore_barrier()
        cpa = pltpu.async_copy(m_sh.at[i_v.at[pl.ds(0, h)]],
                               r_v.at[pl.ds(0, h), :], sem)
        cpb = pltpu.async_copy(m_sh.at[i_v.at[pl.ds(h, h)]],
                               r_v.at[pl.ds(h, h), :], sem2)
        cpa.wait()
        pltpu.sync_copy(r_v.at[pl.ds(0, h), :], o_hbm.at[pl.ds(b, h)])
        cpb.wait()
        pltpu.sync_copy(r_v.at[pl.ds(h, h), :], o_hbm.at[pl.ds(b + h, h)])

    return functools.partial(
        pl.kernel,
        out_type=jax.ShapeDtypeStruct((n_out, R), jnp.float32),
        mesh=plsc.VectorSubcoreMesh(core_axis_name="c",
                                    subcore_axis_name="s"),
        scratch_types=[
            pltpu.VMEM_SHARED((s_rows, R), jnp.float32),
            pltpu.VMEM((c,), jnp.int32),
            pltpu.VMEM((c,), jnp.int32),
            pltpu.VMEM((c, R), jnp.float32),
            pltpu.SemaphoreType.DMA,
            pltpu.SemaphoreType.DMA,
            pltpu.SemaphoreType.DMA,
        ],
        compiler_params=pltpu.CompilerParams(use_tc_tiling_on_sc=False,
                                             needs_layout_passes=False),
    )(body)


_sc_gather1 = _make_sc_gather(S1)
_sc_gather2 = _make_sc_gather(S2)


_BI = 1024


def _prep_body(w2_ref, r2_ref, g_ref, b8_ref):
    acc = w2_ref[:, 0:R] * r2_ref[:, 0:R]
    for k in range(1, K):
        acc = acc + (w2_ref[:, k * R:(k + 1) * R]
                     * r2_ref[:, k * R:(k + 1) * R])
    bt = lax.dot_general(
        g_ref[...], acc, (((1,), (1,)), ((), ())),
        preferred_element_type=jnp.float32)
    for k in range(K):
        b8_ref[k * R:(k + 1) * R, :] = bt


def _main_body(w1_ref, r1_ref, b8_ref, out_ref):
    p = w1_ref[...] * r1_ref[...]
    out_ref[...] = jnp.dot(p, b8_ref[...],
                           preferred_element_type=jnp.float32)


def kernel(x, M1, M2, G, idx1, idx2, dist1, dist2):
    del x
    r2 = _sc_gather2(idx2.T.reshape(-1), M2)
    r1 = _sc_gather1(idx1.T.reshape(-1), M1)
    r1f = r1.reshape(S1, K * R)
    r2f = r2.reshape(S2, K * R)
    we1 = jnp.repeat(dist1, R, axis=1)
    we2 = jnp.repeat(dist2, R, axis=1)

    b8 = pl.pallas_call(
        _prep_body,
        out_shape=jax.ShapeDtypeStruct((K * R, S2), jnp.float32),
    )(we2, r2f, G)

    out = pl.pallas_call(
        _main_body,
        grid=(S1 // _BI,),
        in_specs=[
            pl.BlockSpec((_BI, K * R), lambda i: (i, 0)),
            pl.BlockSpec((_BI, K * R), lambda i: (i, 0)),
            pl.BlockSpec((K * R, S2), lambda i: (0, 0)),
        ],
        out_specs=pl.BlockSpec((_BI, S2), lambda i: (i, 0)),
        out_shape=jax.ShapeDtypeStruct((S1, S2), jnp.float32),
        compiler_params=pltpu.CompilerParams(
            dimension_semantics=("parallel",)),
    )(we1, r1f, b8)
    return out
